# baseline (speedup 1.0000x reference)
import functools

import jax
import jax.numpy as jnp
from jax import lax
from jax.experimental import pallas as pl
from jax.experimental.pallas import tpu as pltpu
from jax.experimental.pallas import tpu_sc as plsc

C = 7
CPAD = 8
A = 128
LANES = 16
IGNORE = 255


def _sc_partials(feat, labels, n_workers, rows_per_w, chunk):
    n_chunks = rows_per_w // chunk
    mesh = plsc.VectorSubcoreMesh(core_axis_name="c", subcore_axis_name="s")

    tec_rows = 240
    sc_rows_c = chunk - tec_rows
    batch = 80
    n_batch = sc_rows_c // batch
    n_sub = 16

    @functools.partial(
        pl.kernel,
        mesh=mesh,
        out_type=(
            jax.ShapeDtypeStruct((n_workers, CPAD, A), jnp.float32),
            jax.ShapeDtypeStruct((n_workers, CPAD, A), jnp.float32),
            jax.ShapeDtypeStruct((n_workers, CPAD, LANES), jnp.float32),
        ),
        scratch_types=[
            pltpu.VMEM((chunk, A), jnp.float32),
            pltpu.VMEM((chunk, A), jnp.float32),
            pltpu.VMEM((chunk,), jnp.int32),
            pltpu.VMEM((chunk,), jnp.int32),
            pltpu.VMEM((n_batch, batch), jnp.int32),
            pltpu.VMEM((n_batch, batch), jnp.int32),
            pltpu.VMEM((CPAD, A), jnp.float32),
            pltpu.VMEM((CPAD, LANES), jnp.float32),
            pltpu.VMEM_SHARED((n_sub * CPAD, A), jnp.float32),
            pltpu.SemaphoreType.DMA,
            pltpu.SemaphoreType.DMA,
            pltpu.SemaphoreType.DMA,
            pltpu.SemaphoreType.DMA,
            pltpu.SemaphoreType.DMA,
            pltpu.SemaphoreType.DMA,
        ],
    )
    def body(feat_hbm, labels_hbm, sums_out, sums2_out, counts_out,
             fbuf0, fbuf1, lbuf0, lbuf1, ibuf0, ibuf1, acc, cacc, shared,
             fsem0, fsem1, lsem0, lsem1, ssem0, ssem1):
        n_cores = 2
        sid = lax.axis_index("s")
        wid = sid * n_cores + lax.axis_index("c")
        row0 = wid * rows_per_w
        region0 = sid * CPAD

        fbufs = (fbuf0, fbuf1)
        lbufs = (lbuf0, lbuf1)
        ibufs = (ibuf0, ibuf1)
        fsems = (fsem0, fsem1)
        lsems = (lsem0, lsem1)
        ssems = (ssem0, ssem1)

        def start_in(ci, slot):
            base = row0 + ci * chunk
            pltpu.make_async_copy(
                feat_hbm.at[pl.ds(base, chunk)], fbufs[slot], fsems[slot]
            ).start()
            pltpu.make_async_copy(
                labels_hbm.at[pl.ds(base, chunk)], lbufs[slot], lsems[slot]
            ).start()

        def wait_in(slot):
            pltpu.make_async_copy(
                feat_hbm.at[pl.ds(0, chunk)], fbufs[slot], fsems[slot]
            ).wait()
            pltpu.make_async_copy(
                labels_hbm.at[pl.ds(0, chunk)], lbufs[slot], lsems[slot]
            ).wait()

        zero = jnp.zeros((LANES,), jnp.float32)
        for r in range(CPAD):
            for k in range(A // LANES):
                acc[r, pl.ds(k * LANES, LANES)] = zero
        pltpu.sync_copy(acc, shared.at[pl.ds(region0, CPAD)])

        base_vec = jnp.full((LANES,), CPAD, dtype=jnp.int32) * sid
        sevenv = jnp.full((LANES,), C, dtype=jnp.uint32)
        one = jnp.ones((LANES,), jnp.float32)
        zf = jnp.zeros((LANES,), jnp.float32)

        def scat(slot, j):
            return pltpu.async_copy(
                fbufs[slot].at[pl.ds(tec_rows + j * batch, batch)],
                shared.at[ibufs[slot].at[j]],
                ssems[slot],
                add=True,
            )

        def consume(slot, cnts):
            lbuf = lbufs[slot]
            ibuf = ibufs[slot]
            fbuf = fbufs[slot]
            for g in range(chunk // LANES):
                lab16 = lbuf[pl.ds(g * LANES, LANES)]
                cnts = tuple(
                    cnts[c] + jnp.where(lab16 == c, one, zf)
                    for c in range(C)
                )
                if g * LANES >= tec_rows:
                    lab_u = jnp.minimum(
                        lab16.astype(jnp.uint32), sevenv).astype(jnp.int32)
                    idx = lab_u + base_vec
                    off = g * LANES - tec_rows
                    ibuf[off // batch, pl.ds(off % batch, LANES)] = idx
            descs = [scat(slot, j) for j in range(n_batch)]

            def group_body(g, _):
                lab16 = lbuf[pl.ds(g * LANES, LANES)]
                labs = []
                for r in range(LANES):
                    label = lab16[r]
                    valid = jnp.logical_and(label >= 0, label < C)
                    labs.append(jnp.where(valid, label, C))

                def load_row(r):
                    row = g * LANES + r
                    return [fbuf[row, pl.ds(k * LANES, LANES)]
                            for k in range(A // LANES)]

                vals = load_row(0)
                for r in range(LANES):
                    nxt = []
                    for k in range(A // LANES):
                        plsc.addupdate(
                            acc.at[labs[r], pl.ds(k * LANES, LANES)], vals[k])
                        if r + 1 < LANES:
                            nxt.append(fbuf[g * LANES + r + 1,
                                            pl.ds(k * LANES, LANES)])
                    vals = nxt
                return 0

            lax.fori_loop(0, tec_rows // LANES, group_body, 0)
            for d in descs:
                d.wait()
            return cnts

        start_in(0, 0)
        start_in(1, 1)
        n_pairs = n_chunks // 2
        zcnt = jnp.zeros((LANES,), jnp.float32)

        def pair_body(p, cnts):
            ci0 = 2 * p
            wait_in(0)
            cnts = consume(0, cnts)

            @pl.when(ci0 + 2 < n_chunks)
            def _():
                start_in(ci0 + 2, 0)

            wait_in(1)
            cnts = consume(1, cnts)

            @pl.when(ci0 + 3 < n_chunks)
            def _():
                start_in(ci0 + 3, 1)

            return cnts

        cnts = lax.fori_loop(0, n_pairs, pair_body, (zcnt,) * C)
        if n_chunks % 2 == 1:
            wait_in(0)
            cnts = consume(0, cnts)

        for c in range(C):
            cacc[c, :] = cnts[c]
        cacc[C, :] = zf

        pltpu.sync_copy(shared.at[pl.ds(region0, CPAD)], sums_out.at[wid])
        pltpu.sync_copy(acc, sums2_out.at[wid])
        pltpu.sync_copy(cacc, counts_out.at[wid])

    return body(feat, labels)


def _tc_segment_body(feat_ref, labels_ref, sums_ref, counts_ref):
    labs = labels_ref[0, 0, :]
    oh = (labs[:, None] ==
          lax.broadcasted_iota(jnp.int32, (labs.shape[0], CPAD), 1)
          ).astype(jnp.float32)
    contrib = lax.dot_general(
        oh, feat_ref[...], (((0,), (0,)), ((), ())),
        preferred_element_type=jnp.float32)
    cnt = jnp.broadcast_to(jnp.sum(oh, axis=0)[:, None], (CPAD, A))

    @pl.when(pl.program_id(0) == 0)
    def _():
        sums_ref[...] = contrib
        counts_ref[...] = cnt

    @pl.when(pl.program_id(0) != 0)
    def _():
        sums_ref[...] += contrib
        counts_ref[...] += cnt


def _tc_segment(feat, labels, tc_row0, blk):
    n = feat.shape[0]
    n_blocks = (n - tc_row0) // blk
    off = tc_row0 // blk
    labels3 = labels.reshape(n // blk, 1, blk)
    return pl.pallas_call(
        _tc_segment_body,
        grid=(n_blocks,),
        in_specs=[
            pl.BlockSpec((blk, A), lambda i: (off + i, 0)),
            pl.BlockSpec((1, 1, blk), lambda i: (off + i, 0, 0)),
        ],
        out_specs=(
            pl.BlockSpec((CPAD, A), lambda i: (0, 0)),
            pl.BlockSpec((CPAD, A), lambda i: (0, 0)),
        ),
        out_shape=(
            jax.ShapeDtypeStruct((CPAD, A), jnp.float32),
            jax.ShapeDtypeStruct((CPAD, A), jnp.float32),
        ),
    )(feat, labels3)


def _tc_finish_body(proto_ref, sums_ref, sums2_ref, counts_ref,
                    tsums_ref, tcounts_ref, loss_ref, mean_ref):
    sums = (jnp.sum(sums_ref[...], axis=0) + jnp.sum(sums2_ref[...], axis=0)
            + tsums_ref[...])
    counts = (jnp.sum(counts_ref[...], axis=(0, 2))
              + tcounts_ref[..., 0])[:, None]
    denom = jnp.where(counts == 0.0, 1.0, counts)
    mean = (sums / denom)[:C]
    mean_ref[...] = mean

    proto = proto_ref[...]
    pn = proto / jnp.maximum(
        jnp.sqrt(jnp.sum(proto * proto, axis=1, keepdims=True)), 1e-12)
    cn = mean / jnp.maximum(
        jnp.sqrt(jnp.sum(mean * mean, axis=1, keepdims=True)), 1e-12)
    logits = lax.dot_general(
        cn, pn, (((1,), (1,)), ((), ())), preferred_element_type=jnp.float32)

    row_sum = jnp.sum(logits, axis=1)
    valid = row_sum != 0.0
    m = jnp.max(logits, axis=1)
    lse = jnp.log(jnp.sum(jnp.exp(logits - m[:, None]), axis=1)) + m
    eye = (lax.broadcasted_iota(jnp.int32, (C, C), 0)
           == lax.broadcasted_iota(jnp.int32, (C, C), 1))
    diag = jnp.sum(jnp.where(eye, logits, 0.0), axis=1)
    ce = lse - diag
    num = jnp.sum(valid.astype(jnp.int32))
    loss = jnp.sum(jnp.where(valid, ce, 0.0)) / jnp.maximum(num, 1)
    loss_ref[...] = jnp.reshape(loss, (1, 1))


def kernel(Proto, feat, labels):
    n = feat.shape[0]
    n_workers = 32
    sc_rows = 192000
    blk = 4000
    rows_per_w = sc_rows // n_workers
    chunk = 400

    sums, sums2, counts = _sc_partials(feat, labels, n_workers, rows_per_w,
                                       chunk)
    tsums, tcounts = _tc_segment(feat, labels, sc_rows, blk)

    loss2d, mean = pl.pallas_call(
        _tc_finish_body,
        out_shape=(
            jax.ShapeDtypeStruct((1, 1), jnp.float32),
            jax.ShapeDtypeStruct((C, A), jnp.float32),
        ),
    )(Proto, sums, sums2, counts, tsums, tcounts)
    return (loss2d[0, 0], mean)


# speedup vs baseline: 1.0019x; 1.0019x over previous
import functools

import jax
import jax.numpy as jnp
from jax import lax
from jax.experimental import pallas as pl
from jax.experimental.pallas import tpu as pltpu
from jax.experimental.pallas import tpu_sc as plsc

C = 7
CPAD = 8
A = 128
LANES = 16
IGNORE = 255


def _sc_partials(feat, labels, n_workers, rows_per_w, chunk):
    n_chunks = rows_per_w // chunk
    mesh = plsc.VectorSubcoreMesh(core_axis_name="c", subcore_axis_name="s")

    batch = 80
    n_batch = chunk // batch
    n_sub = 16

    @functools.partial(
        pl.kernel,
        mesh=mesh,
        out_type=(
            jax.ShapeDtypeStruct((n_workers, CPAD, A), jnp.float32),
            jax.ShapeDtypeStruct((n_workers, CPAD, LANES), jnp.float32),
        ),
        scratch_types=[
            pltpu.VMEM((chunk, A), jnp.float32),
            pltpu.VMEM((chunk, A), jnp.float32),
            pltpu.VMEM((chunk,), jnp.int32),
            pltpu.VMEM((chunk,), jnp.int32),
            pltpu.VMEM((n_batch, batch), jnp.int32),
            pltpu.VMEM((n_batch, batch), jnp.int32),
            pltpu.VMEM((CPAD, A), jnp.float32),
            pltpu.VMEM((CPAD, LANES), jnp.float32),
            pltpu.VMEM_SHARED((n_sub * CPAD, A), jnp.float32),
            pltpu.SemaphoreType.DMA,
            pltpu.SemaphoreType.DMA,
            pltpu.SemaphoreType.DMA,
            pltpu.SemaphoreType.DMA,
            pltpu.SemaphoreType.DMA,
            pltpu.SemaphoreType.DMA,
        ],
    )
    def body(feat_hbm, labels_hbm, sums_out, counts_out,
             fbuf0, fbuf1, lbuf0, lbuf1, ibuf0, ibuf1, zbuf, cacc, shared,
             fsem0, fsem1, lsem0, lsem1, ssem0, ssem1):
        n_cores = 2
        sid = lax.axis_index("s")
        wid = sid * n_cores + lax.axis_index("c")
        row0 = wid * rows_per_w
        region0 = sid * CPAD

        fbufs = (fbuf0, fbuf1)
        lbufs = (lbuf0, lbuf1)
        ibufs = (ibuf0, ibuf1)
        fsems = (fsem0, fsem1)
        lsems = (lsem0, lsem1)
        ssems = (ssem0, ssem1)

        def start_in(ci, slot):
            base = row0 + ci * chunk
            pltpu.make_async_copy(
                feat_hbm.at[pl.ds(base, chunk)], fbufs[slot], fsems[slot]
            ).start()
            pltpu.make_async_copy(
                labels_hbm.at[pl.ds(base, chunk)], lbufs[slot], lsems[slot]
            ).start()

        def wait_in(slot):
            pltpu.make_async_copy(
                feat_hbm.at[pl.ds(0, chunk)], fbufs[slot], fsems[slot]
            ).wait()
            pltpu.make_async_copy(
                labels_hbm.at[pl.ds(0, chunk)], lbufs[slot], lsems[slot]
            ).wait()

        zero = jnp.zeros((LANES,), jnp.float32)
        for r in range(CPAD):
            for k in range(A // LANES):
                zbuf[r, pl.ds(k * LANES, LANES)] = zero
        pltpu.sync_copy(zbuf, shared.at[pl.ds(region0, CPAD)])

        base_vec = jnp.full((LANES,), CPAD, dtype=jnp.int32) * sid
        sevenv = jnp.full((LANES,), C, dtype=jnp.uint32)
        one = jnp.ones((LANES,), jnp.float32)
        zf = jnp.zeros((LANES,), jnp.float32)

        def scat(slot, j):
            return pltpu.async_copy(
                fbufs[slot].at[pl.ds(j * batch, batch)],
                shared.at[ibufs[slot].at[j]],
                ssems[slot],
                add=True,
            )

        def consume(slot, cnts):
            lbuf = lbufs[slot]
            ibuf = ibufs[slot]
            for g in range(chunk // LANES):
                lab16 = lbuf[pl.ds(g * LANES, LANES)]
                cnts = tuple(
                    cnts[c] + jnp.where(lab16 == c, one, zf)
                    for c in range(C)
                )
                lab_u = jnp.minimum(
                    lab16.astype(jnp.uint32), sevenv).astype(jnp.int32)
                idx = lab_u + base_vec
                b = (g * LANES) // batch
                off = (g * LANES) % batch
                ibuf[b, pl.ds(off, LANES)] = idx
            descs = [scat(slot, j) for j in range(n_batch)]
            for d in descs:
                d.wait()
            return cnts

        start_in(0, 0)
        start_in(1, 1)
        n_pairs = n_chunks // 2
        zcnt = jnp.zeros((LANES,), jnp.float32)

        def pair_body(p, cnts):
            ci0 = 2 * p
            wait_in(0)
            cnts = consume(0, cnts)

            @pl.when(ci0 + 2 < n_chunks)
            def _():
                start_in(ci0 + 2, 0)

            wait_in(1)
            cnts = consume(1, cnts)

            @pl.when(ci0 + 3 < n_chunks)
            def _():
                start_in(ci0 + 3, 1)

            return cnts

        cnts = lax.fori_loop(0, n_pairs, pair_body, (zcnt,) * C)
        if n_chunks % 2 == 1:
            wait_in(0)
            cnts = consume(0, cnts)

        for c in range(C):
            cacc[c, :] = cnts[c]
        cacc[C, :] = zf

        pltpu.sync_copy(shared.at[pl.ds(region0, CPAD)], sums_out.at[wid])
        pltpu.sync_copy(cacc, counts_out.at[wid])

    return body(feat, labels)


def _tc_segment_body(feat_a, labels_a, feat_b, labels_b, sums_ref, counts_ref):
    def contrib_of(feat_ref, labels_ref):
        labs = labels_ref[0, 0, :]
        oh = (labs[:, None] ==
              lax.broadcasted_iota(jnp.int32, (labs.shape[0], CPAD), 1)
              ).astype(jnp.float32)
        contrib = lax.dot_general(
            oh, feat_ref[...], (((0,), (0,)), ((), ())),
            preferred_element_type=jnp.float32)
        cnt = jnp.broadcast_to(jnp.sum(oh, axis=0)[:, None], (CPAD, A))
        return contrib, cnt

    ca, cnta = contrib_of(feat_a, labels_a)
    cb, cntb = contrib_of(feat_b, labels_b)

    @pl.when(pl.program_id(0) == 0)
    def _():
        sums_ref[...] = ca + cb
        counts_ref[...] = cnta + cntb

    @pl.when(pl.program_id(0) != 0)
    def _():
        sums_ref[...] += ca + cb
        counts_ref[...] += cnta + cntb


def _tc_segment(feat, labels, tc_row0, blk):
    n = feat.shape[0]
    n_blocks = (n - tc_row0) // blk
    n_steps = n_blocks // 2
    off = tc_row0 // blk
    labels3 = labels.reshape(n // blk, 1, blk)
    return pl.pallas_call(
        _tc_segment_body,
        grid=(n_steps,),
        in_specs=[
            pl.BlockSpec((blk, A), lambda i: (off + 2 * i, 0)),
            pl.BlockSpec((1, 1, blk), lambda i: (off + 2 * i, 0, 0)),
            pl.BlockSpec((blk, A), lambda i: (off + 2 * i + 1, 0)),
            pl.BlockSpec((1, 1, blk), lambda i: (off + 2 * i + 1, 0, 0)),
        ],
        out_specs=(
            pl.BlockSpec((CPAD, A), lambda i: (0, 0)),
            pl.BlockSpec((CPAD, A), lambda i: (0, 0)),
        ),
        out_shape=(
            jax.ShapeDtypeStruct((CPAD, A), jnp.float32),
            jax.ShapeDtypeStruct((CPAD, A), jnp.float32),
        ),
    )(feat, labels3, feat, labels3)


def _tc_finish_body(proto_ref, sums_ref, counts_ref, tsums_ref, tcounts_ref,
                    loss_ref, mean_ref):
    sums = jnp.sum(sums_ref[...], axis=0) + tsums_ref[...]
    counts = (jnp.sum(counts_ref[...], axis=(0, 2))
              + tcounts_ref[..., 0])[:, None]
    denom = jnp.where(counts == 0.0, 1.0, counts)
    mean = (sums / denom)[:C]
    mean_ref[...] = mean

    proto = proto_ref[...]
    pn = proto / jnp.maximum(
        jnp.sqrt(jnp.sum(proto * proto, axis=1, keepdims=True)), 1e-12)
    cn = mean / jnp.maximum(
        jnp.sqrt(jnp.sum(mean * mean, axis=1, keepdims=True)), 1e-12)
    logits = lax.dot_general(
        cn, pn, (((1,), (1,)), ((), ())), preferred_element_type=jnp.float32)

    row_sum = jnp.sum(logits, axis=1)
    valid = row_sum != 0.0
    m = jnp.max(logits, axis=1)
    lse = jnp.log(jnp.sum(jnp.exp(logits - m[:, None]), axis=1)) + m
    eye = (lax.broadcasted_iota(jnp.int32, (C, C), 0)
           == lax.broadcasted_iota(jnp.int32, (C, C), 1))
    diag = jnp.sum(jnp.where(eye, logits, 0.0), axis=1)
    ce = lse - diag
    num = jnp.sum(valid.astype(jnp.int32))
    loss = jnp.sum(jnp.where(valid, ce, 0.0)) / jnp.maximum(num, 1)
    loss_ref[...] = jnp.reshape(loss, (1, 1))


def kernel(Proto, feat, labels):
    n = feat.shape[0]
    n_workers = 32
    sc_rows = 192000
    blk = 4000
    rows_per_w = sc_rows // n_workers
    chunk = 400

    sums, counts = _sc_partials(feat, labels, n_workers, rows_per_w, chunk)
    tsums, tcounts = _tc_segment(feat, labels, sc_rows, blk)

    loss2d, mean = pl.pallas_call(
        _tc_finish_body,
        out_shape=(
            jax.ShapeDtypeStruct((1, 1), jnp.float32),
            jax.ShapeDtypeStruct((C, A), jnp.float32),
        ),
    )(Proto, sums, counts, tsums, tcounts)
    return (loss2d[0, 0], mean)


# speedup vs baseline: 1.0434x; 1.0414x over previous
import functools

import jax
import jax.numpy as jnp
from jax import lax
from jax.experimental import pallas as pl
from jax.experimental.pallas import tpu as pltpu
from jax.experimental.pallas import tpu_sc as plsc

C = 7
CPAD = 8
A = 128
LANES = 16
IGNORE = 255


def _sc_partials(feat, labels, n_workers, rows_per_w, chunk):
    n_chunks = rows_per_w // chunk
    mesh = plsc.VectorSubcoreMesh(core_axis_name="c", subcore_axis_name="s")

    batch = 80
    n_batch = chunk // batch
    n_sub = 16

    @functools.partial(
        pl.kernel,
        mesh=mesh,
        out_type=(
            jax.ShapeDtypeStruct((n_workers, CPAD, A), jnp.float32),
            jax.ShapeDtypeStruct((n_workers, CPAD, LANES), jnp.float32),
        ),
        scratch_types=[
            pltpu.VMEM((chunk, A), jnp.float32),
            pltpu.VMEM((chunk, A), jnp.float32),
            pltpu.VMEM((chunk,), jnp.int32),
            pltpu.VMEM((chunk,), jnp.int32),
            pltpu.VMEM((n_batch, batch), jnp.int32),
            pltpu.VMEM((n_batch, batch), jnp.int32),
            pltpu.VMEM((CPAD, A), jnp.float32),
            pltpu.VMEM((CPAD, LANES), jnp.float32),
            pltpu.VMEM_SHARED((n_sub * CPAD, A), jnp.float32),
            pltpu.SemaphoreType.DMA,
            pltpu.SemaphoreType.DMA,
            pltpu.SemaphoreType.DMA,
            pltpu.SemaphoreType.DMA,
            pltpu.SemaphoreType.DMA,
            pltpu.SemaphoreType.DMA,
        ],
    )
    def body(feat_hbm, labels_hbm, sums_out, counts_out,
             fbuf0, fbuf1, lbuf0, lbuf1, ibuf0, ibuf1, zbuf, cacc, shared,
             fsem0, fsem1, lsem0, lsem1, ssem0, ssem1):
        n_cores = 2
        sid = lax.axis_index("s")
        wid = sid * n_cores + lax.axis_index("c")
        row0 = wid * rows_per_w
        region0 = sid * CPAD

        fbufs = (fbuf0, fbuf1)
        lbufs = (lbuf0, lbuf1)
        ibufs = (ibuf0, ibuf1)
        fsems = (fsem0, fsem1)
        lsems = (lsem0, lsem1)
        ssems = (ssem0, ssem1)

        n_sub_dma = 5
        sub = chunk // n_sub_dma

        def start_in(ci, slot):
            base = row0 + ci * chunk
            for q in range(n_sub_dma):
                pltpu.make_async_copy(
                    feat_hbm.at[pl.ds(base + q * sub, sub)],
                    fbufs[slot].at[pl.ds(q * sub, sub)],
                    fsems[slot],
                ).start()
            pltpu.make_async_copy(
                labels_hbm.at[pl.ds(base, chunk)], lbufs[slot], lsems[slot]
            ).start()

        def wait_in(slot):
            pltpu.make_async_copy(
                feat_hbm.at[pl.ds(0, chunk)], fbufs[slot], fsems[slot]
            ).wait()
            pltpu.make_async_copy(
                labels_hbm.at[pl.ds(0, chunk)], lbufs[slot], lsems[slot]
            ).wait()

        zero = jnp.zeros((LANES,), jnp.float32)
        for r in range(CPAD):
            for k in range(A // LANES):
                zbuf[r, pl.ds(k * LANES, LANES)] = zero
        pltpu.sync_copy(zbuf, shared.at[pl.ds(region0, CPAD)])

        base_vec = jnp.full((LANES,), CPAD, dtype=jnp.int32) * sid
        sevenv = jnp.full((LANES,), C, dtype=jnp.uint32)
        one = jnp.ones((LANES,), jnp.float32)
        zf = jnp.zeros((LANES,), jnp.float32)

        def scat(slot, j):
            return pltpu.async_copy(
                fbufs[slot].at[pl.ds(j * batch, batch)],
                shared.at[ibufs[slot].at[j]],
                ssems[slot],
                add=True,
            )

        def consume(slot, cnts):
            lbuf = lbufs[slot]
            ibuf = ibufs[slot]
            for g in range(chunk // LANES):
                lab16 = lbuf[pl.ds(g * LANES, LANES)]
                cnts = tuple(
                    cnts[c] + jnp.where(lab16 == c, one, zf)
                    for c in range(C)
                )
                lab_u = jnp.minimum(
                    lab16.astype(jnp.uint32), sevenv).astype(jnp.int32)
                idx = lab_u + base_vec
                b = (g * LANES) // batch
                off = (g * LANES) % batch
                ibuf[b, pl.ds(off, LANES)] = idx
            descs = [scat(slot, j) for j in range(n_batch)]
            for d in descs:
                d.wait()
            return cnts

        start_in(0, 0)
        start_in(1, 1)
        n_pairs = n_chunks // 2
        zcnt = jnp.zeros((LANES,), jnp.float32)

        def pair_body(p, cnts):
            ci0 = 2 * p
            wait_in(0)
            cnts = consume(0, cnts)

            @pl.when(ci0 + 2 < n_chunks)
            def _():
                start_in(ci0 + 2, 0)

            wait_in(1)
            cnts = consume(1, cnts)

            @pl.when(ci0 + 3 < n_chunks)
            def _():
                start_in(ci0 + 3, 1)

            return cnts

        cnts = lax.fori_loop(0, n_pairs, pair_body, (zcnt,) * C)
        if n_chunks % 2 == 1:
            wait_in(0)
            cnts = consume(0, cnts)

        for c in range(C):
            cacc[c, :] = cnts[c]
        cacc[C, :] = zf

        pltpu.sync_copy(shared.at[pl.ds(region0, CPAD)], sums_out.at[wid])
        pltpu.sync_copy(cacc, counts_out.at[wid])

    return body(feat, labels)


def _tc_segment_body(feat_ref, labels_ref, sums_ref, counts_ref):
    labs = labels_ref[0, 0, :]
    oh = (labs[:, None] ==
          lax.broadcasted_iota(jnp.int32, (labs.shape[0], CPAD), 1)
          ).astype(jnp.float32)
    contrib = lax.dot_general(
        oh, feat_ref[...], (((0,), (0,)), ((), ())),
        preferred_element_type=jnp.float32)
    cnt = jnp.broadcast_to(jnp.sum(oh, axis=0)[:, None], (CPAD, A))

    @pl.when(pl.program_id(0) == 0)
    def _():
        sums_ref[...] = contrib
        counts_ref[...] = cnt

    @pl.when(pl.program_id(0) != 0)
    def _():
        sums_ref[...] += contrib
        counts_ref[...] += cnt


def _tc_segment(feat, labels, tc_row0, blk):
    n = feat.shape[0]
    n_blocks = (n - tc_row0) // blk
    off = tc_row0 // blk
    labels3 = labels.reshape(n // blk, 1, blk)
    return pl.pallas_call(
        _tc_segment_body,
        grid=(n_blocks,),
        in_specs=[
            pl.BlockSpec((blk, A), lambda i: (off + i, 0)),
            pl.BlockSpec((1, 1, blk), lambda i: (off + i, 0, 0)),
        ],
        out_specs=(
            pl.BlockSpec((CPAD, A), lambda i: (0, 0)),
            pl.BlockSpec((CPAD, A), lambda i: (0, 0)),
        ),
        out_shape=(
            jax.ShapeDtypeStruct((CPAD, A), jnp.float32),
            jax.ShapeDtypeStruct((CPAD, A), jnp.float32),
        ),
    )(feat, labels3)


def _tc_finish_body(proto_ref, sums_ref, counts_ref, tsums_ref, tcounts_ref,
                    loss_ref, mean_ref):
    sums = jnp.sum(sums_ref[...], axis=0) + tsums_ref[...]
    counts = (jnp.sum(counts_ref[...], axis=(0, 2))
              + tcounts_ref[..., 0])[:, None]
    denom = jnp.where(counts == 0.0, 1.0, counts)
    mean = (sums / denom)[:C]
    mean_ref[...] = mean

    proto = proto_ref[...]
    pn = proto / jnp.maximum(
        jnp.sqrt(jnp.sum(proto * proto, axis=1, keepdims=True)), 1e-12)
    cn = mean / jnp.maximum(
        jnp.sqrt(jnp.sum(mean * mean, axis=1, keepdims=True)), 1e-12)
    logits = lax.dot_general(
        cn, pn, (((1,), (1,)), ((), ())), preferred_element_type=jnp.float32)

    row_sum = jnp.sum(logits, axis=1)
    valid = row_sum != 0.0
    m = jnp.max(logits, axis=1)
    lse = jnp.log(jnp.sum(jnp.exp(logits - m[:, None]), axis=1)) + m
    eye = (lax.broadcasted_iota(jnp.int32, (C, C), 0)
           == lax.broadcasted_iota(jnp.int32, (C, C), 1))
    diag = jnp.sum(jnp.where(eye, logits, 0.0), axis=1)
    ce = lse - diag
    num = jnp.sum(valid.astype(jnp.int32))
    loss = jnp.sum(jnp.where(valid, ce, 0.0)) / jnp.maximum(num, 1)
    loss_ref[...] = jnp.reshape(loss, (1, 1))


def kernel(Proto, feat, labels):
    n = feat.shape[0]
    n_workers = 32
    sc_rows = 192000
    blk = 4000
    rows_per_w = sc_rows // n_workers
    chunk = 400

    sums, counts = _sc_partials(feat, labels, n_workers, rows_per_w, chunk)
    tsums, tcounts = _tc_segment(feat, labels, sc_rows, blk)

    loss2d, mean = pl.pallas_call(
        _tc_finish_body,
        out_shape=(
            jax.ShapeDtypeStruct((1, 1), jnp.float32),
            jax.ShapeDtypeStruct((C, A), jnp.float32),
        ),
    )(Proto, sums, counts, tsums, tcounts)
    return (loss2d[0, 0], mean)


# speedup vs baseline: 1.0478x; 1.0042x over previous
import functools

import jax
import jax.numpy as jnp
from jax import lax
from jax.experimental import pallas as pl
from jax.experimental.pallas import tpu as pltpu
from jax.experimental.pallas import tpu_sc as plsc

C = 7
CPAD = 8
A = 128
LANES = 16
IGNORE = 255


def _sc_partials(feat, labels, n_workers, rows_per_w, chunk):
    n_chunks = rows_per_w // chunk
    mesh = plsc.VectorSubcoreMesh(core_axis_name="c", subcore_axis_name="s")

    batch = 80
    n_batch = chunk // batch
    n_sub = 16

    @functools.partial(
        pl.kernel,
        mesh=mesh,
        out_type=(
            jax.ShapeDtypeStruct((n_workers, CPAD, A), jnp.float32),
            jax.ShapeDtypeStruct((n_workers, CPAD, LANES), jnp.float32),
        ),
        scratch_types=[
            pltpu.VMEM((chunk, A), jnp.float32),
            pltpu.VMEM((chunk, A), jnp.float32),
            pltpu.VMEM((chunk,), jnp.int32),
            pltpu.VMEM((chunk,), jnp.int32),
            pltpu.VMEM((n_batch, batch), jnp.int32),
            pltpu.VMEM((n_batch, batch), jnp.int32),
            pltpu.VMEM((CPAD, A), jnp.float32),
            pltpu.VMEM((CPAD, LANES), jnp.float32),
            pltpu.VMEM_SHARED((n_sub * CPAD, A), jnp.float32),
            pltpu.SemaphoreType.DMA,
            pltpu.SemaphoreType.DMA,
            pltpu.SemaphoreType.DMA,
            pltpu.SemaphoreType.DMA,
            pltpu.SemaphoreType.DMA,
            pltpu.SemaphoreType.DMA,
        ],
    )
    def body(feat_hbm, labels_hbm, sums_out, counts_out,
             fbuf0, fbuf1, lbuf0, lbuf1, ibuf0, ibuf1, zbuf, cacc, shared,
             fsem0, fsem1, lsem0, lsem1, ssem0, ssem1):
        n_cores = 2
        sid = lax.axis_index("s")
        wid = sid * n_cores + lax.axis_index("c")
        row0 = wid * rows_per_w
        region0 = sid * CPAD

        fbufs = (fbuf0, fbuf1)
        lbufs = (lbuf0, lbuf1)
        ibufs = (ibuf0, ibuf1)
        fsems = (fsem0, fsem1)
        lsems = (lsem0, lsem1)
        ssems = (ssem0, ssem1)

        def start_in(ci, slot):
            base = row0 + ci * chunk
            pltpu.make_async_copy(
                feat_hbm.at[pl.ds(base, chunk)], fbufs[slot], fsems[slot]
            ).start()
            pltpu.make_async_copy(
                labels_hbm.at[pl.ds(base, chunk)], lbufs[slot], lsems[slot]
            ).start()

        def wait_in(slot):
            pltpu.make_async_copy(
                feat_hbm.at[pl.ds(0, chunk)], fbufs[slot], fsems[slot]
            ).wait()
            pltpu.make_async_copy(
                labels_hbm.at[pl.ds(0, chunk)], lbufs[slot], lsems[slot]
            ).wait()

        zero = jnp.zeros((LANES,), jnp.float32)
        for r in range(CPAD):
            for k in range(A // LANES):
                zbuf[r, pl.ds(k * LANES, LANES)] = zero
        pltpu.sync_copy(zbuf, shared.at[pl.ds(region0, CPAD)])

        base_vec = jnp.full((LANES,), CPAD, dtype=jnp.int32) * sid
        sevenv = jnp.full((LANES,), C, dtype=jnp.uint32)
        one = jnp.ones((LANES,), jnp.float32)
        zf = jnp.zeros((LANES,), jnp.float32)

        def scat(slot, j):
            return pltpu.async_copy(
                fbufs[slot].at[pl.ds(j * batch, batch)],
                shared.at[ibufs[slot].at[j]],
                ssems[slot],
                add=True,
            )

        def consume(slot, cnts):
            lbuf = lbufs[slot]
            ibuf = ibufs[slot]
            for g in range(chunk // LANES):
                lab16 = lbuf[pl.ds(g * LANES, LANES)]
                cnts = tuple(
                    cnts[c] + jnp.where(lab16 == c, one, zf)
                    for c in range(C)
                )
                lab_u = jnp.minimum(
                    lab16.astype(jnp.uint32), sevenv).astype(jnp.int32)
                idx = lab_u + base_vec
                b = (g * LANES) // batch
                off = (g * LANES) % batch
                ibuf[b, pl.ds(off, LANES)] = idx
            descs = [scat(slot, j) for j in range(n_batch)]
            for d in descs:
                d.wait()
            return cnts

        start_in(0, 0)
        start_in(1, 1)
        n_pairs = n_chunks // 2
        zcnt = jnp.zeros((LANES,), jnp.float32)

        def pair_body(p, cnts):
            ci0 = 2 * p
            wait_in(0)
            cnts = consume(0, cnts)

            @pl.when(ci0 + 2 < n_chunks)
            def _():
                start_in(ci0 + 2, 0)

            wait_in(1)
            cnts = consume(1, cnts)

            @pl.when(ci0 + 3 < n_chunks)
            def _():
                start_in(ci0 + 3, 1)

            return cnts

        cnts = lax.fori_loop(0, n_pairs, pair_body, (zcnt,) * C)
        if n_chunks % 2 == 1:
            wait_in(0)
            cnts = consume(0, cnts)

        for c in range(C):
            cacc[c, :] = cnts[c]
        cacc[C, :] = zf

        pltpu.sync_copy(shared.at[pl.ds(region0, CPAD)], sums_out.at[wid])
        pltpu.sync_copy(cacc, counts_out.at[wid])

    return body(feat, labels)


def _tc_segment_body(feat_ref, labels_ref, sums_ref, counts_ref):
    labs = labels_ref[0, 0, :]
    oh = (labs[:, None] ==
          lax.broadcasted_iota(jnp.int32, (labs.shape[0], CPAD), 1)
          ).astype(jnp.float32)
    contrib = lax.dot_general(
        oh, feat_ref[...], (((0,), (0,)), ((), ())),
        preferred_element_type=jnp.float32)
    cnt = jnp.broadcast_to(jnp.sum(oh, axis=0)[:, None], (CPAD, A))

    @pl.when(pl.program_id(0) == 0)
    def _():
        sums_ref[...] = contrib
        counts_ref[...] = cnt

    @pl.when(pl.program_id(0) != 0)
    def _():
        sums_ref[...] += contrib
        counts_ref[...] += cnt


def _tc_segment(feat, labels, tc_row0, blk):
    n = feat.shape[0]
    n_blocks = (n - tc_row0) // blk
    off = tc_row0 // blk
    labels3 = labels.reshape(n // blk, 1, blk)
    return pl.pallas_call(
        _tc_segment_body,
        grid=(n_blocks,),
        in_specs=[
            pl.BlockSpec((blk, A), lambda i: (off + i, 0)),
            pl.BlockSpec((1, 1, blk), lambda i: (off + i, 0, 0)),
        ],
        out_specs=(
            pl.BlockSpec((CPAD, A), lambda i: (0, 0)),
            pl.BlockSpec((CPAD, A), lambda i: (0, 0)),
        ),
        out_shape=(
            jax.ShapeDtypeStruct((CPAD, A), jnp.float32),
            jax.ShapeDtypeStruct((CPAD, A), jnp.float32),
        ),
    )(feat, labels3)


def _tc_finish_body(proto_ref, sums_ref, counts_ref, tsums_ref, tcounts_ref,
                    loss_ref, mean_ref):
    sums = jnp.sum(sums_ref[...], axis=0) + tsums_ref[...]
    counts = (jnp.sum(counts_ref[...], axis=(0, 2))
              + tcounts_ref[..., 0])[:, None]
    denom = jnp.where(counts == 0.0, 1.0, counts)
    mean = (sums / denom)[:C]
    mean_ref[...] = mean

    proto = proto_ref[...]
    pn = proto / jnp.maximum(
        jnp.sqrt(jnp.sum(proto * proto, axis=1, keepdims=True)), 1e-12)
    cn = mean / jnp.maximum(
        jnp.sqrt(jnp.sum(mean * mean, axis=1, keepdims=True)), 1e-12)
    logits = lax.dot_general(
        cn, pn, (((1,), (1,)), ((), ())), preferred_element_type=jnp.float32)

    row_sum = jnp.sum(logits, axis=1)
    valid = row_sum != 0.0
    m = jnp.max(logits, axis=1)
    lse = jnp.log(jnp.sum(jnp.exp(logits - m[:, None]), axis=1)) + m
    eye = (lax.broadcasted_iota(jnp.int32, (C, C), 0)
           == lax.broadcasted_iota(jnp.int32, (C, C), 1))
    diag = jnp.sum(jnp.where(eye, logits, 0.0), axis=1)
    ce = lse - diag
    num = jnp.sum(valid.astype(jnp.int32))
    loss = jnp.sum(jnp.where(valid, ce, 0.0)) / jnp.maximum(num, 1)
    loss_ref[...] = jnp.reshape(loss, (1, 1))


def kernel(Proto, feat, labels):
    n = feat.shape[0]
    n_workers = 32
    sc_rows = 192000
    blk = 4000
    rows_per_w = sc_rows // n_workers
    chunk = 400

    sums, counts = _sc_partials(feat, labels, n_workers, rows_per_w, chunk)
    tsums, tcounts = _tc_segment(feat, labels, sc_rows, blk)

    loss2d, mean = pl.pallas_call(
        _tc_finish_body,
        out_shape=(
            jax.ShapeDtypeStruct((1, 1), jnp.float32),
            jax.ShapeDtypeStruct((C, A), jnp.float32),
        ),
    )(Proto, sums, counts, tsums, tcounts)
    return (loss2d[0, 0], mean)
